# baseline (speedup 1.0000x reference)
import functools

import jax
import jax.numpy as jnp
from jax.experimental import pallas as pl
from jax.experimental.pallas import tpu as pltpu
from jax.experimental.pallas import tpu_sc as plsc

_F32 = jnp.float32


def _shift_rows(a, k):
    L, D = a.shape
    z = jnp.zeros((abs(k), D), a.dtype)
    if k > 0:
        return jnp.concatenate([a[k:], z], axis=0)
    return jnp.concatenate([z, a[:k]], axis=0)


def _conv3(xb, w_ref, b_row):
    a0 = jnp.dot(xb, w_ref[0], preferred_element_type=_F32)
    a1 = jnp.dot(xb, w_ref[1], preferred_element_type=_F32)
    a2 = jnp.dot(xb, w_ref[2], preferred_element_type=_F32)
    return _shift_rows(a0, -1) + a1 + _shift_rows(a2, 1) + b_row


def _layernorm(h, g_row, b_row, ones_col):
    d = h.shape[1]
    s1 = jnp.dot(h, ones_col, preferred_element_type=_F32)
    s2 = jnp.dot(h * h, ones_col, preferred_element_type=_F32)
    m = s1 * (1.0 / d)
    v = s2 * (1.0 / d) - m * m
    rv = jax.lax.rsqrt(v + 1e-5)
    return (h - m) * rv * g_row + b_row


def _predictor(xb, w1_ref, w2_ref, v_ref, lw_col, ones_col):
    h = _conv3(xb, w1_ref, v_ref[0:1, :])
    h = jnp.maximum(h, 0.0)
    h = _layernorm(h, v_ref[1:2, :], v_ref[2:3, :], ones_col)
    h = _conv3(h, w2_ref, v_ref[3:4, :])
    h = jnp.maximum(h, 0.0)
    h = _layernorm(h, v_ref[4:5, :], v_ref[5:6, :], ones_col)
    return jnp.dot(h, lw_col, preferred_element_type=_F32) + v_ref[7:8, 0:1]


def _vecpack(p, D):
    return jnp.stack([
        p["c1b"], p["ln1g"], p["ln1b"],
        p["c2b"], p["ln2g"], p["ln2b"],
        p["lw"][:, 0], jnp.broadcast_to(p["lb"], (D,)),
    ], axis=0).astype(_F32)


def _idx_body(dur_ref, tri_ref, ml_ref, gidx_ref):
    S = dur_ref.shape[2]
    T = gidx_ref.shape[1]
    b = pl.program_id(0)
    dur_row = dur_ref[0]
    cum_f = jnp.dot(dur_row, tri_ref[...], preferred_element_type=_F32)
    cum_i = (cum_f + 0.5).astype(jnp.int32)
    pos_i = jax.lax.broadcasted_iota(jnp.int32, (T, 1), 0)
    cmp = jnp.where(cum_i <= pos_i, 1.0, 0.0)
    ones_col = jnp.full((S, 1), 1.0, _F32)
    cnt = jnp.dot(cmp, ones_col, preferred_element_type=_F32)
    cnt_i = (cnt + 0.5).astype(jnp.int32)
    valid = jnp.logical_and(cnt_i < S, pos_i < ml_ref[0:1, 0:1])
    zrow = gidx_ref.shape[0] * 0 + _ZROW_SENTINEL
    gidx_ref[0] = jnp.where(valid, b * S + cnt_i, zrow)


_ZROW_SENTINEL = 0


def _make_idx_body(zrow):
    def body(dur_ref, tri_ref, ml_ref, gidx_ref):
        S = dur_ref.shape[2]
        T = gidx_ref.shape[1]
        b = pl.program_id(0)
        dur_row = dur_ref[0]
        cum_f = jnp.dot(dur_row, tri_ref[...], preferred_element_type=_F32)
        cum_i = (cum_f + 0.5).astype(jnp.int32)
        pos_i = jax.lax.broadcasted_iota(jnp.int32, (T, 1), 0)
        cmp = jnp.where(cum_i <= pos_i, 1.0, 0.0)
        ones_col = jnp.full((S, 1), 1.0, _F32)
        cnt = jnp.dot(cmp, ones_col, preferred_element_type=_F32)
        cnt_i = (cnt + 0.5).astype(jnp.int32)
        valid = jnp.logical_and(cnt_i < S, pos_i < ml_ref[0:1, 0:1])
        gidx_ref[0] = jnp.where(valid, b * S + cnt_i, zrow)
    return body


def _dur_body(x_ref, dw1, dw2, dv, lw_ref, ld_ref):
    xb = x_ref[0]
    ones_col = jnp.full((xb.shape[1], 1), 1.0, _F32)
    ld_ref[0] = _predictor(xb, dw1, dw2, dv, lw_ref[:, 0:1], ones_col)


def _sc_gather(x_flat_pad, gidx_flat):
    NROWS = gidx_flat.shape[0]
    D = x_flat_pad.shape[1]
    NC, NS = 2, 16
    NW = NC * NS
    per_w = NROWS // NW
    CH = 128
    nch = per_w // CH
    mesh = plsc.VectorSubcoreMesh(
        core_axis_name="c", subcore_axis_name="s",
        num_cores=NC, num_subcores=NS)

    NSLOT = 3

    @functools.partial(
        pl.kernel, mesh=mesh,
        out_type=jax.ShapeDtypeStruct((NROWS, D), jnp.float32),
        scratch_types=(
            [pltpu.VMEM((CH,), jnp.int32) for _ in range(NSLOT)]
            + [pltpu.VMEM((CH, D), jnp.float32) for _ in range(NSLOT)]
            + [pltpu.SemaphoreType.DMA for _ in range(2 * NSLOT)]
        ),
    )
    def k(x_hbm, gidx_hbm, out_hbm, *scr):
        idxb = scr[0:NSLOT]
        rowb = scr[NSLOT:2 * NSLOT]
        gsem = scr[2 * NSLOT:3 * NSLOT]
        wsem = scr[3 * NSLOT:4 * NSLOT]
        wid = jax.lax.axis_index("s") * NC + jax.lax.axis_index("c")
        base = wid * per_w

        def stage(c):
            s = c % NSLOT
            pltpu.sync_copy(gidx_hbm.at[pl.ds(base + c * CH, CH)], idxb[s])
            return pltpu.async_copy(x_hbm.at[idxb[s]], rowb[s], gsem[s])

        g = {}
        wb = {}
        g[0] = stage(0)
        for c in range(nch):
            nxt = c + 1
            if nxt < nch:
                if nxt >= NSLOT:
                    wb[nxt - NSLOT].wait()
                g[nxt] = stage(nxt)
            g[c].wait()
            wb[c] = pltpu.async_copy(
                rowb[c % NSLOT], out_hbm.at[pl.ds(base + c * CH, CH)],
                wsem[c % NSLOT])
        for c in range(max(0, nch - NSLOT), nch):
            wb[c].wait()

    return k(x_flat_pad, gidx_flat)


def _tail_body(xe_ref, pt_ref, et_ref, bins2_ref, lw_ref,
               pw1, pw2, pv, ew1, ew2, ev, ptab_ref, etab_ref,
               out_ref, pp_ref, ep_ref):
    x_exp = xe_ref[0]
    ones_col = jnp.full((x_exp.shape[1], 1), 1.0, _F32)
    pp_ref[0] = _predictor(x_exp, pw1, pw2, pv, lw_ref[:, 1:2], ones_col)
    ep_ref[0] = _predictor(x_exp, ew1, ew2, ev, lw_ref[:, 2:3], ones_col)
    lo = bins2_ref[0:1, :]
    hi = bins2_ref[1:2, :]
    vp = pt_ref[0]
    ve = et_ref[0]
    p_oh = jnp.where(lo < vp, 1.0, 0.0) - jnp.where(hi < vp, 1.0, 0.0)
    e_oh = jnp.where(lo < ve, 1.0, 0.0) - jnp.where(hi < ve, 1.0, 0.0)
    pemb = jnp.dot(p_oh, ptab_ref[...], preferred_element_type=_F32)
    eemb = jnp.dot(e_oh, etab_ref[...], preferred_element_type=_F32)
    out_ref[0] = x_exp + pemb + eemb


def kernel(x, pitch_target, energy_target, params, src_mask, mel_mask,
           duration_target, max_len):
    B, S, D = x.shape
    T = mel_mask.shape[1]

    dur_f = duration_target.astype(_F32).reshape(B, 1, S)
    pt_col = pitch_target.reshape(B, T, 1)
    et_col = energy_target.reshape(B, T, 1)
    bins = jnp.linspace(0.0, 1.0, 255, dtype=_F32)
    bins_lo = jnp.concatenate([jnp.full((1,), -1e30, _F32), bins])
    bins_hi = jnp.concatenate([bins, jnp.full((1,), 1e30, _F32)])
    bins2 = jnp.stack([bins_lo, bins_hi], axis=0)
    ii = jax.lax.broadcasted_iota(jnp.int32, (S, S), 0)
    jj = jax.lax.broadcasted_iota(jnp.int32, (S, S), 1)
    tri = jnp.where(ii <= jj, 1.0, 0.0).astype(_F32)
    ml = jnp.broadcast_to(jnp.asarray(max_len, jnp.int32), (1, 1))

    dp, pp_, ep_ = params["dur"], params["pitch"], params["energy"]
    lw_cols = jnp.concatenate([
        dp["lw"], pp_["lw"], ep_["lw"],
        jnp.zeros((D, 5), _F32)], axis=1)

    batch = lambda *blk: pl.BlockSpec(blk, lambda b: (b,) + (0,) * (len(blk) - 1))
    bcast = lambda *blk: pl.BlockSpec(blk, lambda b: (0,) * len(blk))
    wspecs = [bcast(3, D, D), bcast(3, D, D), bcast(8, D)]

    zrow = B * S
    gidx = pl.pallas_call(
        _make_idx_body(zrow),
        grid=(B,),
        in_specs=[batch(1, 1, S), bcast(S, S), bcast(1, 1)],
        out_specs=batch(1, T, 1),
        out_shape=jax.ShapeDtypeStruct((B, T, 1), jnp.int32),
        compiler_params=pltpu.CompilerParams(
            dimension_semantics=("parallel",)),
    )(dur_f, tri, ml)

    ld = pl.pallas_call(
        _dur_body,
        grid=(B,),
        in_specs=[batch(1, S, D), *wspecs, bcast(D, 8)],
        out_specs=batch(1, S, 1),
        out_shape=jax.ShapeDtypeStruct((B, S, 1), _F32),
        compiler_params=pltpu.CompilerParams(
            dimension_semantics=("parallel",)),
    )(x, dp["c1w"], dp["c2w"], _vecpack(dp, D), lw_cols)

    x_flat_pad = jnp.concatenate(
        [x.reshape(B * S, D), jnp.zeros((8, D), _F32)], axis=0)
    xexp = _sc_gather(x_flat_pad, gidx.reshape(B * T)).reshape(B, T, D)

    out, pp_col, ep_col = pl.pallas_call(
        _tail_body,
        grid=(B,),
        in_specs=[
            batch(1, T, D), batch(1, T, 1), batch(1, T, 1),
            bcast(2, 256), bcast(D, 8),
            *wspecs, *wspecs,
            bcast(256, D), bcast(256, D),
        ],
        out_specs=[batch(1, T, D), batch(1, T, 1), batch(1, T, 1)],
        out_shape=[
            jax.ShapeDtypeStruct((B, T, D), _F32),
            jax.ShapeDtypeStruct((B, T, 1), _F32),
            jax.ShapeDtypeStruct((B, T, 1), _F32),
        ],
        compiler_params=pltpu.CompilerParams(
            dimension_semantics=("parallel",)),
    )(xexp, pt_col, et_col, bins2, lw_cols,
      pp_["c1w"], pp_["c2w"], _vecpack(pp_, D),
      ep_["c1w"], ep_["c2w"], _vecpack(ep_, D),
      params["pitch_table"], params["energy_table"])

    log_dur = jnp.where(src_mask, 0.0, ld.reshape(B, S))
    pitch_pred = jnp.where(mel_mask, 0.0, pp_col.reshape(B, T))
    energy_pred = jnp.where(mel_mask, 0.0, ep_col.reshape(B, T))
    return (out, log_dur, pitch_pred, energy_pred), (duration_target, mel_mask)


# speedup vs baseline: 1.8045x; 1.8045x over previous
import jax
import jax.numpy as jnp
from jax.experimental import pallas as pl
from jax.experimental.pallas import tpu as pltpu

_F32 = jnp.float32
_NT = 2
_HALO = 2


def _shift_rows(a, k):
    L, D = a.shape
    z = jnp.zeros((abs(k), D), a.dtype)
    if k > 0:
        return jnp.concatenate([a[k:], z], axis=0)
    return jnp.concatenate([z, a[:k]], axis=0)


def _conv3(xb, w_ref, b_row):
    a0 = jnp.dot(xb, w_ref[0], preferred_element_type=_F32)
    a1 = jnp.dot(xb, w_ref[1], preferred_element_type=_F32)
    a2 = jnp.dot(xb, w_ref[2], preferred_element_type=_F32)
    return _shift_rows(a0, -1) + a1 + _shift_rows(a2, 1) + b_row


def _layernorm(h, g_row, b_row, ones_col):
    d = h.shape[1]
    s1 = jnp.dot(h, ones_col, preferred_element_type=_F32)
    s2 = jnp.dot(h * h, ones_col, preferred_element_type=_F32)
    m = s1 * (1.0 / d)
    v = s2 * (1.0 / d) - m * m
    rv = jax.lax.rsqrt(v + 1e-5)
    return (h - m) * rv * g_row + b_row


def _predictor(xb, w1_ref, w2_ref, v_ref, lw_col, ones_col, row_mask=None):
    h = _conv3(xb, w1_ref, v_ref[0:1, :])
    h = jnp.maximum(h, 0.0)
    h = _layernorm(h, v_ref[1:2, :], v_ref[2:3, :], ones_col)
    if row_mask is not None:
        h = h * row_mask
    h = _conv3(h, w2_ref, v_ref[3:4, :])
    h = jnp.maximum(h, 0.0)
    h = _layernorm(h, v_ref[4:5, :], v_ref[5:6, :], ones_col)
    return jnp.dot(h, lw_col, preferred_element_type=_F32) + v_ref[7:8, 0:1]


def _body(x_ref, dur_ref, pt_ref, et_ref, bins2_ref, tri_ref, lw_ref,
          dw1, dw2, dv, pw1, pw2, pv, ew1, ew2, ev,
          ptab_ref, etab_ref, ml_ref,
          out_ref, ld_ref, pp_ref, ep_ref):
    S = x_ref.shape[1]
    TT = pt_ref.shape[1]
    t = pl.program_id(1)
    xb = x_ref[0]
    ones_col = jnp.full((xb.shape[1], 1), 1.0, _F32)

    @pl.when(t == 0)
    def _():
        ld_ref[0] = _predictor(xb, dw1, dw2, dv, lw_ref[:, 0:1], ones_col)

    dur_row = dur_ref[0]
    cum_f = jnp.dot(dur_row, tri_ref[...], preferred_element_type=_F32)
    cum_i = (cum_f + 0.5).astype(jnp.int32)
    dur_i = (dur_row + 0.5).astype(jnp.int32)
    cum_prev_i = cum_i - dur_i
    LE = TT + 2 * _HALO
    pos_i = (jax.lax.broadcasted_iota(jnp.int32, (LE, 1), 0)
             + t * TT - _HALO)
    ok = jnp.logical_and(pos_i >= 0, pos_i < ml_ref[0:1, 0:1])
    pos_i = jnp.where(ok, pos_i, -1)
    onehot = (jnp.where(cum_prev_i <= pos_i, 1.0, 0.0)
              - jnp.where(cum_i <= pos_i, 1.0, 0.0))
    x_ext = jnp.dot(onehot, xb, preferred_element_type=_F32)

    row_mask = jnp.where(pos_i >= 0, 1.0, 0.0)
    pp_ref[0] = _predictor(x_ext, pw1, pw2, pv, lw_ref[:, 1:2],
                           ones_col, row_mask)[_HALO:TT + _HALO]
    ep_ref[0] = _predictor(x_ext, ew1, ew2, ev, lw_ref[:, 2:3],
                           ones_col, row_mask)[_HALO:TT + _HALO]

    lo = bins2_ref[0:1, :]
    hi = bins2_ref[1:2, :]
    vp = pt_ref[0]
    ve = et_ref[0]
    p_oh = jnp.where(lo < vp, 1.0, 0.0) - jnp.where(hi < vp, 1.0, 0.0)
    e_oh = jnp.where(lo < ve, 1.0, 0.0) - jnp.where(hi < ve, 1.0, 0.0)
    pemb = jnp.dot(p_oh, ptab_ref[...], preferred_element_type=_F32)
    eemb = jnp.dot(e_oh, etab_ref[...], preferred_element_type=_F32)

    out_ref[0] = x_ext[_HALO:TT + _HALO] + pemb + eemb


def _vecpack(p, D):
    return jnp.stack([
        p["c1b"], p["ln1g"], p["ln1b"],
        p["c2b"], p["ln2g"], p["ln2b"],
        p["lw"][:, 0], jnp.broadcast_to(p["lb"], (D,)),
    ], axis=0).astype(_F32)


def kernel(x, pitch_target, energy_target, params, src_mask, mel_mask,
           duration_target, max_len):
    B, S, D = x.shape
    T = mel_mask.shape[1]
    TT = T // _NT

    dur_f = duration_target.astype(_F32).reshape(B, 1, S)
    pt_col = pitch_target.reshape(B, T, 1)
    et_col = energy_target.reshape(B, T, 1)
    bins = jnp.linspace(0.0, 1.0, 255, dtype=_F32)
    bins_lo = jnp.concatenate([jnp.full((1,), -1e30, _F32), bins])
    bins_hi = jnp.concatenate([bins, jnp.full((1,), 1e30, _F32)])
    bins2 = jnp.stack([bins_lo, bins_hi], axis=0)
    ii = jax.lax.broadcasted_iota(jnp.int32, (S, S), 0)
    jj = jax.lax.broadcasted_iota(jnp.int32, (S, S), 1)
    tri = jnp.where(ii <= jj, 1.0, 0.0).astype(_F32)
    ml = jnp.broadcast_to(jnp.asarray(max_len, jnp.int32), (1, 1))

    dp, pp_, ep_ = params["dur"], params["pitch"], params["energy"]
    lw_cols = jnp.concatenate([
        dp["lw"], pp_["lw"], ep_["lw"],
        jnp.zeros((D, 5), _F32)], axis=1)
    operands = (
        x, dur_f, pt_col, et_col, bins2, tri, lw_cols,
        dp["c1w"], dp["c2w"], _vecpack(dp, D),
        pp_["c1w"], pp_["c2w"], _vecpack(pp_, D),
        ep_["c1w"], ep_["c2w"], _vecpack(ep_, D),
        params["pitch_table"], params["energy_table"], ml,
    )

    bb = lambda *blk: pl.BlockSpec(blk, lambda b, t: (b,) + (0,) * (len(blk) - 1))
    bt = lambda *blk: pl.BlockSpec(blk, lambda b, t: (b, t, 0))
    bcast = lambda *blk: pl.BlockSpec(blk, lambda b, t: (0,) * len(blk))
    wspecs = [bcast(3, D, D), bcast(3, D, D), bcast(8, D)]
    in_specs = [
        bb(1, S, D), bb(1, 1, S), bt(1, TT, 1), bt(1, TT, 1),
        bcast(2, 256), bcast(S, S), bcast(D, 8),
        *wspecs, *wspecs, *wspecs,
        bcast(256, D), bcast(256, D), bcast(1, 1),
    ]
    out_specs = [bt(1, TT, D), bb(1, S, 1), bt(1, TT, 1), bt(1, TT, 1)]
    out_shape = [
        jax.ShapeDtypeStruct((B, T, D), _F32),
        jax.ShapeDtypeStruct((B, S, 1), _F32),
        jax.ShapeDtypeStruct((B, T, 1), _F32),
        jax.ShapeDtypeStruct((B, T, 1), _F32),
    ]

    out, ld, pp_col, ep_col = pl.pallas_call(
        _body,
        grid=(B, _NT),
        in_specs=in_specs,
        out_specs=out_specs,
        out_shape=out_shape,
        compiler_params=pltpu.CompilerParams(
            dimension_semantics=("parallel", "arbitrary")),
    )(*operands)

    log_dur = jnp.where(src_mask, 0.0, ld.reshape(B, S))
    pitch_pred = jnp.where(mel_mask, 0.0, pp_col.reshape(B, T))
    energy_pred = jnp.where(mel_mask, 0.0, ep_col.reshape(B, T))
    return (out, log_dur, pitch_pred, energy_pred), (duration_target, mel_mask)


# speedup vs baseline: 1.9988x; 1.1077x over previous
import jax
import jax.numpy as jnp
from jax.experimental import pallas as pl
from jax.experimental.pallas import tpu as pltpu

_F32 = jnp.float32


def _shift_rows(a, k):
    L, D = a.shape
    z = jnp.zeros((abs(k), D), a.dtype)
    if k > 0:
        return jnp.concatenate([a[k:], z], axis=0)
    return jnp.concatenate([z, a[:k]], axis=0)


def _conv3(xb, w_ref, b_row):
    a0 = jnp.dot(xb, w_ref[0], preferred_element_type=_F32)
    a1 = jnp.dot(xb, w_ref[1], preferred_element_type=_F32)
    a2 = jnp.dot(xb, w_ref[2], preferred_element_type=_F32)
    return _shift_rows(a0, -1) + a1 + _shift_rows(a2, 1) + b_row


def _layernorm(h, g_row, b_row, ones_col):
    d = h.shape[1]
    s1 = jnp.dot(h, ones_col, preferred_element_type=_F32)
    s2 = jnp.dot(h * h, ones_col, preferred_element_type=_F32)
    m = s1 * (1.0 / d)
    v = s2 * (1.0 / d) - m * m
    rv = jax.lax.rsqrt(v + 1e-5)
    return (h - m) * rv * g_row + b_row


def _predictor(xb, w1_ref, w2_ref, v_ref, lw_col, ones_col):
    h = _conv3(xb, w1_ref, v_ref[0:1, :])
    h = jnp.maximum(h, 0.0)
    h = _layernorm(h, v_ref[1:2, :], v_ref[2:3, :], ones_col)
    h = _conv3(h, w2_ref, v_ref[3:4, :])
    h = jnp.maximum(h, 0.0)
    h = _layernorm(h, v_ref[4:5, :], v_ref[5:6, :], ones_col)
    out = jnp.dot(h, lw_col, preferred_element_type=_F32) + v_ref[7:8, 0:1]
    return out


def _body(x_ref, dur_ref, pt_ref, et_ref, bins2_ref, tri_ref, lw_ref,
          dw1, dw2, dv, pw1, pw2, pv, ew1, ew2, ev,
          ptab_ref, etab_ref, ml_ref,
          out_ref, ld_ref, pp_ref, ep_ref):
    S = x_ref.shape[1]
    T = pt_ref.shape[1]
    xb = x_ref[0]
    ones_col = jnp.full((xb.shape[1], 1), 1.0, _F32)

    ld_ref[0] = _predictor(xb, dw1, dw2, dv, lw_ref[:, 0:1], ones_col)

    dur_row = dur_ref[0]
    cum_f = jnp.dot(dur_row, tri_ref[...], preferred_element_type=_F32)
    cum_i = (cum_f + 0.5).astype(jnp.int32)
    dur_i = (dur_row + 0.5).astype(jnp.int32)
    cum_prev_i = cum_i - dur_i
    pos_i = jax.lax.broadcasted_iota(jnp.int32, (T, 1), 0)
    pos_i = jnp.where(pos_i < ml_ref[0:1, 0:1], pos_i, -1)
    onehot = (jnp.where(cum_prev_i <= pos_i, 1.0, 0.0)
              - jnp.where(cum_i <= pos_i, 1.0, 0.0))
    x_exp = jnp.dot(onehot, xb, preferred_element_type=_F32)

    pp_ref[0] = _predictor(x_exp, pw1, pw2, pv, lw_ref[:, 1:2], ones_col)
    ep_ref[0] = _predictor(x_exp, ew1, ew2, ev, lw_ref[:, 2:3], ones_col)

    lo = bins2_ref[0:1, :]
    hi = bins2_ref[1:2, :]
    vp = pt_ref[0]
    ve = et_ref[0]
    p_oh = jnp.where(lo < vp, 1.0, 0.0) - jnp.where(hi < vp, 1.0, 0.0)
    e_oh = jnp.where(lo < ve, 1.0, 0.0) - jnp.where(hi < ve, 1.0, 0.0)
    pemb = jnp.dot(p_oh, ptab_ref[...], preferred_element_type=_F32)
    eemb = jnp.dot(e_oh, etab_ref[...], preferred_element_type=_F32)

    out_ref[0] = x_exp + pemb + eemb


def _vecpack(p, D):
    return jnp.stack([
        p["c1b"], p["ln1g"], p["ln1b"],
        p["c2b"], p["ln2g"], p["ln2b"],
        p["lw"][:, 0], jnp.broadcast_to(p["lb"], (D,)),
    ], axis=0).astype(_F32)


def kernel(x, pitch_target, energy_target, params, src_mask, mel_mask,
           duration_target, max_len):
    B, S, D = x.shape
    T = mel_mask.shape[1]

    dur_f = duration_target.astype(_F32).reshape(B, 1, S)
    pt_col = pitch_target.reshape(B, T, 1)
    et_col = energy_target.reshape(B, T, 1)
    bins = jnp.linspace(0.0, 1.0, 255, dtype=_F32)
    bins_lo = jnp.concatenate([jnp.full((1,), -1e30, _F32), bins])
    bins_hi = jnp.concatenate([bins, jnp.full((1,), 1e30, _F32)])
    bins2 = jnp.stack([bins_lo, bins_hi], axis=0)
    ii = jax.lax.broadcasted_iota(jnp.int32, (S, S), 0)
    jj = jax.lax.broadcasted_iota(jnp.int32, (S, S), 1)
    tri = jnp.where(ii <= jj, 1.0, 0.0).astype(_F32)
    ml = jnp.broadcast_to(jnp.asarray(max_len, jnp.int32), (1, 1))

    dp, pp_, ep_ = params["dur"], params["pitch"], params["energy"]
    lw_cols = jnp.concatenate([
        dp["lw"], pp_["lw"], ep_["lw"],
        jnp.zeros((D, 5), _F32)], axis=1)
    operands = (
        x, dur_f, pt_col, et_col, bins2, tri, lw_cols,
        dp["c1w"], dp["c2w"], _vecpack(dp, D),
        pp_["c1w"], pp_["c2w"], _vecpack(pp_, D),
        ep_["c1w"], ep_["c2w"], _vecpack(ep_, D),
        params["pitch_table"], params["energy_table"], ml,
    )

    batch = lambda *blk: pl.BlockSpec(blk, lambda b: (b,) + (0,) * (len(blk) - 1))
    bcast = lambda *blk: pl.BlockSpec(blk, lambda b: (0,) * len(blk))
    wspecs = [bcast(3, D, D), bcast(3, D, D), bcast(8, D)]
    in_specs = [
        batch(1, S, D), batch(1, 1, S), batch(1, T, 1), batch(1, T, 1),
        bcast(2, 256), bcast(S, S), bcast(D, 8),
        *wspecs, *wspecs, *wspecs,
        bcast(256, D), bcast(256, D), bcast(1, 1),
    ]
    out_specs = [batch(1, T, D), batch(1, S, 1), batch(1, T, 1), batch(1, T, 1)]
    out_shape = [
        jax.ShapeDtypeStruct((B, T, D), _F32),
        jax.ShapeDtypeStruct((B, S, 1), _F32),
        jax.ShapeDtypeStruct((B, T, 1), _F32),
        jax.ShapeDtypeStruct((B, T, 1), _F32),
    ]

    out, ld, pp_col, ep_col = pl.pallas_call(
        _body,
        grid=(B,),
        in_specs=in_specs,
        out_specs=out_specs,
        out_shape=out_shape,
        compiler_params=pltpu.CompilerParams(
            dimension_semantics=("parallel",)),
    )(*operands)

    log_dur = jnp.where(src_mask, 0.0, ld.reshape(B, S))
    pitch_pred = jnp.where(mel_mask, 0.0, pp_col.reshape(B, T))
    energy_pred = jnp.where(mel_mask, 0.0, ep_col.reshape(B, T))
    return (out, log_dur, pitch_pred, energy_pred), (duration_target, mel_mask)
